# baseline (speedup 1.0000x reference)
import jax
import jax.numpy as jnp
from jax.experimental import pallas as pl
from jax.experimental.pallas import tpu as pltpu

D_FF = 11008
D_MODEL = 4096
BK = 688


def _tc_body(t_ref, x_ref, wn_ref, o_ref, w_ref, y_ref):
    i = pl.program_id(0)
    xb = x_ref[...]
    m = jnp.abs(xb) * wn_ref[...] >= t_ref[0]
    xm = jnp.where(m, xb, jnp.float32(0.0))
    part = jnp.sum(w_ref[...] * xm, axis=0, keepdims=True)

    @pl.when(i == 0)
    def _():
        y_ref[...] = o_ref[...] + part

    @pl.when(i > 0)
    def _():
        y_ref[...] += part


def kernel(x, W_t, W_norm, thresh, out):
    xf = x.reshape(D_FF, 1)
    wn = W_norm.reshape(D_FF, 1)
    t = jnp.reshape(thresh, (1,))
    o = out.reshape(1, D_MODEL)
    nblk = D_FF // BK
    y = pl.pallas_call(
        _tc_body,
        grid=(nblk,),
        in_specs=[
            pl.BlockSpec(memory_space=pltpu.SMEM),
            pl.BlockSpec((BK, 1), lambda i: (i, 0)),
            pl.BlockSpec((BK, 1), lambda i: (i, 0)),
            pl.BlockSpec((1, D_MODEL), lambda i: (0, 0)),
            pl.BlockSpec((BK, D_MODEL), lambda i: (i, 0)),
        ],
        out_specs=pl.BlockSpec((1, D_MODEL), lambda i: (0, 0)),
        out_shape=jax.ShapeDtypeStruct((1, D_MODEL), jnp.float32),
        compiler_params=pltpu.CompilerParams(
            dimension_semantics=("arbitrary",),
        ),
    )(t, xf, wn, o, W_t)
    return y.reshape(D_MODEL)
